# baseline (speedup 1.0000x reference)
import jax
import jax.numpy as jnp
from jax import lax
from jax.experimental import pallas as pl
from jax.experimental.pallas import tpu as pltpu
from jax.experimental.pallas import tpu_sc as plsc

_NC = 2
_NS = 16
_NW = _NC * _NS
_NBUF = 3
_CROWS = 512


def _quant_body(cur_ref, q_ref):
    q_ref[...] = cur_ref[...].astype(jnp.float8_e4m3fn).astype(jnp.float32)


def _sc_body(prev_hbm, curp_hbm, idx_hbm, out_hbm,
             bufs, row_v, idx_v, rs0, rs1, rs2, ws0, ws1, ws2, ssem):
    w = lax.axis_index("s") * _NC + lax.axis_index("c")
    rows_per_w = out_hbm.shape[0] // _NW
    nch = rows_per_w // _CROWS
    base = w * rows_per_w
    rsems = (rs0, rs1, rs2)
    wsems = (ws0, ws1, ws2)

    pltpu.sync_copy(idx_hbm.at[w], idx_v)
    pltpu.sync_copy(curp_hbm.at[w], row_v)

    def rd(i, b):
        r0 = base + i * _CROWS
        return pltpu.async_copy(
            prev_hbm.at[pl.ds(r0, _CROWS)], bufs.at[b], rsems[b])

    def wr(i, b):
        r0 = base + i * _CROWS
        return pltpu.async_copy(
            bufs.at[b], out_hbm.at[pl.ds(r0, _CROWS)], wsems[b])

    for i in range(nch):
        r0 = base + i * _CROWS
        pltpu.sync_copy(prev_hbm.at[pl.ds(r0, _CROWS)], bufs.at[0])
        pltpu.sync_copy(bufs.at[0], out_hbm.at[pl.ds(r0, _CROWS)])

    sc = pltpu.async_copy(row_v, out_hbm.at[idx_v], ssem)
    sc.start()
    sc.wait()


def kernel(prev, cur, dim, idx):
    B, H, S, D = prev.shape
    BH = B * H
    R = BH * S
    per = BH // _NW
    prev2 = prev.reshape(R, D)
    pos = (idx[0] - dim + 1).astype(jnp.int32)
    rowidx = jnp.arange(BH, dtype=jnp.int32) * S + pos
    idx2 = rowidx.reshape(_NW, per)
    idx2 = jnp.concatenate([idx2, idx2], axis=1)
    curp = cur.reshape(_NW, per, D)
    curp = jnp.concatenate([curp, curp], axis=1)

    qcurp = pl.pallas_call(
        _quant_body,
        out_shape=jax.ShapeDtypeStruct(curp.shape, jnp.float32),
    )(curp)

    mesh = plsc.VectorSubcoreMesh(
        core_axis_name="c", subcore_axis_name="s",
        num_cores=_NC, num_subcores=_NS)
    sc_call = pl.kernel(
        _sc_body,
        out_type=jax.ShapeDtypeStruct((R, D), jnp.float32),
        mesh=mesh,
        scratch_types=[
            pltpu.VMEM((1, _CROWS, D), jnp.float32),
            pltpu.VMEM((2 * per, D), jnp.float32),
            pltpu.VMEM((2 * per,), jnp.int32),
            pltpu.SemaphoreType.DMA,
            pltpu.SemaphoreType.DMA,
            pltpu.SemaphoreType.DMA,
            pltpu.SemaphoreType.DMA,
            pltpu.SemaphoreType.DMA,
            pltpu.SemaphoreType.DMA,
            pltpu.SemaphoreType.DMA,
        ],
    )
    out = sc_call(prev2, qcurp, idx2)
    return out.reshape(B, H, S, D)


# speedup vs baseline: 1.4865x; 1.4865x over previous
import jax
import jax.numpy as jnp
from jax.experimental import pallas as pl
from jax.experimental.pallas import tpu as pltpu


def _body(pos_ref, prev_ref, cur_ref, out_ref):
    out_ref[...] = prev_ref[...]
    pos = pos_ref[0]
    q = cur_ref[...].astype(jnp.float8_e4m3fn).astype(out_ref.dtype)
    out_ref[:, pl.ds(pos, 1), :] = q


def kernel(prev, cur, dim, idx):
    B, H, S, D = prev.shape
    BH = B * H
    BLK = 8
    prev3 = prev.reshape(BH, S, D)
    cur3 = cur.reshape(BH, 1, D)
    pos = (idx[0] - dim + 1).astype(jnp.int32).reshape((1,))
    grid_spec = pltpu.PrefetchScalarGridSpec(
        num_scalar_prefetch=1,
        grid=(BH // BLK,),
        in_specs=[
            pl.BlockSpec((BLK, S, D), lambda i, pos_ref: (i, 0, 0)),
            pl.BlockSpec((BLK, 1, D), lambda i, pos_ref: (i, 0, 0)),
        ],
        out_specs=pl.BlockSpec((BLK, S, D), lambda i, pos_ref: (i, 0, 0)),
    )
    out = pl.pallas_call(
        _body,
        grid_spec=grid_spec,
        out_shape=jax.ShapeDtypeStruct((BH, S, D), prev.dtype),
        compiler_params=pltpu.CompilerParams(
            dimension_semantics=("parallel",),
        ),
    )(pos, prev3, cur3)
    return out.reshape(B, H, S, D)
